# baseline (speedup 1.0000x reference)
import jax
import jax.numpy as jnp
from jax.experimental import pallas as pl
from jax.experimental.pallas import tpu as pltpu

_VMEM_LIMIT = 60 * 1024 * 1024


def _pick_tm(rows, limit):
    tm = min(rows, limit)
    while rows % tm:
        tm //= 2
    return tm


def _scatter_counts_tiled(rows_idx, cols_idx, nrows, ncols):
    p = ((rows_idx // 8) * ((ncols // 128) * 8)
         + (cols_idx // 128) * 8 + (rows_idx % 8))
    l = cols_idx % 128
    flat = jnp.zeros((nrows * ncols // 128, 128), jnp.float32).at[p, l].add(1.0)
    return flat.reshape(nrows // 8, ncols // 128, 8, 128)


def _assemble_bf16(m4):
    tm = m4.shape[0] * 8
    nc = m4.shape[1]
    return jnp.concatenate([m4[:, j].reshape(tm, 128) for j in range(nc)],
                           axis=1).astype(jnp.bfloat16)


def _v2e_stage_kernel(m_ref, x_ref, o_ref, s_ref, mb_ref):
    m4 = m_ref[...]
    tm = m4.shape[0] * 8
    m = _assemble_bf16(m4)
    y = jnp.dot(m, x_ref[...], preferred_element_type=jnp.float32)
    o_ref[...] = jnp.maximum(y, 0.0)
    rs = jnp.sum(m4, axis=(1, 3)).reshape(tm, 1)
    s_ref[...] = jnp.broadcast_to(rs, (tm, 128))
    mb_ref[...] = m


def _v2e_stage(m4, x, *, tm):
    nr8, nc, _, _ = m4.shape
    r = nr8 * 8
    k = nc * 128
    d = x.shape[1]
    return pl.pallas_call(
        _v2e_stage_kernel,
        out_shape=(jax.ShapeDtypeStruct((r, d), jnp.float32),
                   jax.ShapeDtypeStruct((r, 128), jnp.float32),
                   jax.ShapeDtypeStruct((r, k), jnp.bfloat16)),
        grid=(r // tm,),
        in_specs=[pl.BlockSpec((tm // 8, nc, 8, 128), lambda i: (i, 0, 0, 0)),
                  pl.BlockSpec((k, d), lambda i: (0, 0))],
        out_specs=(pl.BlockSpec((tm, d), lambda i: (i, 0)),
                   pl.BlockSpec((tm, 128), lambda i: (i, 0)),
                   pl.BlockSpec((tm, k), lambda i: (i, 0))),
        compiler_params=pltpu.CompilerParams(
            dimension_semantics=("parallel",),
            vmem_limit_bytes=_VMEM_LIMIT),
    )(m4, x)


def _e2v_stage_kernel(m_ref, x_ref, o_ref):
    y = jax.lax.dot_general(m_ref[...], x_ref[...], (((0,), (0,)), ((), ())),
                            preferred_element_type=jnp.float32)
    y = jnp.maximum(y, 0.0)
    sq = jnp.sum(y * y, axis=1, keepdims=True)
    o_ref[...] = y * jax.lax.rsqrt(jnp.maximum(sq, 1e-24))


def _e2v_stage(mb, x, *, tm):
    e, r = mb.shape
    d = x.shape[1]
    return pl.pallas_call(
        _e2v_stage_kernel,
        out_shape=jax.ShapeDtypeStruct((r, d), jnp.float32),
        grid=(r // tm,),
        in_specs=[pl.BlockSpec((e, tm), lambda i: (0, i)),
                  pl.BlockSpec((e, d), lambda i: (0, 0))],
        out_specs=pl.BlockSpec((tm, d), lambda i: (i, 0)),
        compiler_params=pltpu.CompilerParams(
            dimension_semantics=("parallel",),
            vmem_limit_bytes=_VMEM_LIMIT),
    )(mb, x)


def _gcn_stage_kernel(m_ref, x_ref, o_ref, mb_ref):
    m = _assemble_bf16(m_ref[...])
    y = jnp.dot(m, x_ref[...], preferred_element_type=jnp.float32)
    y = jnp.maximum(y, 0.0)
    sq = jnp.sum(y * y, axis=1, keepdims=True)
    o_ref[...] = y * jax.lax.rsqrt(jnp.maximum(sq, 1e-24))
    mb_ref[...] = m


def _gcn_stage(m4, x, *, tm):
    nr8, nc, _, _ = m4.shape
    r = nr8 * 8
    k = nc * 128
    d = x.shape[1]
    return pl.pallas_call(
        _gcn_stage_kernel,
        out_shape=(jax.ShapeDtypeStruct((r, d), jnp.float32),
                   jax.ShapeDtypeStruct((r, k), jnp.bfloat16)),
        grid=(r // tm,),
        in_specs=[pl.BlockSpec((tm // 8, nc, 8, 128), lambda i: (i, 0, 0, 0)),
                  pl.BlockSpec((k, d), lambda i: (0, 0))],
        out_specs=(pl.BlockSpec((tm, d), lambda i: (i, 0)),
                   pl.BlockSpec((tm, k), lambda i: (i, 0))),
        compiler_params=pltpu.CompilerParams(
            dimension_semantics=("parallel",),
            vmem_limit_bytes=_VMEM_LIMIT),
    )(m4, x)


def _final_stage_kernel(m_ref, h1b_ref, x_ref, xv_ref, h1_ref, o_ref):
    y = jnp.dot(m_ref[...], h1b_ref[...], preferred_element_type=jnp.float32)
    y = jnp.maximum(y, 0.0)
    sq = jnp.sum(y * y, axis=1, keepdims=True)
    h2 = y * jax.lax.rsqrt(jnp.maximum(sq, 1e-24))

    x = x_ref[...]
    a = xv_ref[...]
    b = h1_ref[...]
    tot = (jnp.sum(x * x, axis=1, keepdims=True)
           + jnp.sum(a * a, axis=1, keepdims=True)
           + jnp.sum(b * b, axis=1, keepdims=True)
           + jnp.sum(h2 * h2, axis=1, keepdims=True))
    inv = jax.lax.rsqrt(jnp.maximum(tot, 1e-24))
    d = x.shape[1]
    o_ref[:, 0:d] = x * inv
    o_ref[:, d:2 * d] = a * inv
    o_ref[:, 2 * d:3 * d] = b * inv
    o_ref[:, 3 * d:4 * d] = h2 * inv


def _final_stage(mb, h1b, x, xv, h1, *, tm):
    r, k = mb.shape
    d = x.shape[1]
    row_spec = pl.BlockSpec((tm, d), lambda i: (i, 0))
    return pl.pallas_call(
        _final_stage_kernel,
        out_shape=jax.ShapeDtypeStruct((r, 4 * d), jnp.float32),
        grid=(r // tm,),
        in_specs=[pl.BlockSpec((tm, k), lambda i: (i, 0)),
                  pl.BlockSpec((k, d), lambda i: (0, 0)),
                  row_spec, row_spec, row_spec],
        out_specs=pl.BlockSpec((tm, 4 * d), lambda i: (i, 0)),
        compiler_params=pltpu.CompilerParams(
            dimension_semantics=("parallel",),
            vmem_limit_bytes=_VMEM_LIMIT),
    )(mb, h1b, x, xv, h1)


def _graph_forward(x, a_t4, b4, size_v, size_e):
    x32 = x.astype(jnp.float32)
    xb = x32.astype(jnp.bfloat16)

    z, zs, a_bf = _v2e_stage(a_t4, xb, tm=_pick_tm(size_e, 256))
    deg_e = zs[:, 0]
    zb = (z * (1.0 / jnp.maximum(deg_e, 1.0))[:, None]).astype(jnp.bfloat16)

    x_v = _e2v_stage(a_bf, zb, tm=_pick_tm(size_v, 1024))
    h1, b_bf = _gcn_stage(b4, xb, tm=_pick_tm(size_v, 256))
    return _final_stage(b_bf, h1.astype(jnp.bfloat16), x32, x_v, h1,
                        tm=_pick_tm(size_v, 512))


@jax.jit
def kernel(x1, x2, edge1, edge2):
    a1 = _scatter_counts_tiled(edge1[1], edge1[0], 2048, 8192)
    a2 = _scatter_counts_tiled(edge2[1], edge2[0], 1536, 6144)
    b1 = _scatter_counts_tiled(edge1[2], edge1[0], 8192, 8192)
    b2 = _scatter_counts_tiled(edge2[2], edge2[0], 6144, 6144)
    y1 = _graph_forward(x1, a1, b1, 8192, 2048)
    y2 = _graph_forward(x2, a2, b2, 6144, 1536)
    return y1, y2


# speedup vs baseline: 1.0487x; 1.0487x over previous
import jax
import jax.numpy as jnp
from jax.experimental import pallas as pl
from jax.experimental.pallas import tpu as pltpu

_VMEM_LIMIT = 60 * 1024 * 1024


def _pick_tm(rows, limit):
    tm = min(rows, limit)
    while rows % tm:
        tm //= 2
    return tm


def _scatter_counts_tiled(rows_idx, cols_idx, nrows, ncols):
    p = ((rows_idx // 8) * ((ncols // 128) * 8)
         + (cols_idx // 128) * 8 + (rows_idx % 8))
    l = cols_idx % 128
    flat = jnp.zeros((nrows * ncols // 128, 128), jnp.float32).at[p, l].add(1.0)
    return flat.reshape(nrows // 8, ncols // 128, 8, 128)


def _assemble_bf16(m4):
    tm = m4.shape[0] * 8
    nc = m4.shape[1]
    return jnp.concatenate([m4[:, j].reshape(tm, 128) for j in range(nc)],
                           axis=1).astype(jnp.bfloat16)


def _v2e_stage_kernel(m_ref, x_ref, o_ref, s_ref):
    m4 = m_ref[...]
    tm = m4.shape[0] * 8
    y = jnp.dot(_assemble_bf16(m4), x_ref[...],
                preferred_element_type=jnp.float32)
    o_ref[...] = jnp.maximum(y, 0.0)
    rs = jnp.sum(m4, axis=(1, 3)).reshape(tm, 1)
    s_ref[...] = jnp.broadcast_to(rs, (tm, 128))


def _v2e_stage(m4, x, *, tm):
    nr8, nc, _, _ = m4.shape
    r = nr8 * 8
    d = x.shape[1]
    return pl.pallas_call(
        _v2e_stage_kernel,
        out_shape=(jax.ShapeDtypeStruct((r, d), jnp.float32),
                   jax.ShapeDtypeStruct((r, 128), jnp.float32)),
        grid=(r // tm,),
        in_specs=[pl.BlockSpec((tm // 8, nc, 8, 128), lambda i: (i, 0, 0, 0)),
                  pl.BlockSpec((nc * 128, d), lambda i: (0, 0))],
        out_specs=(pl.BlockSpec((tm, d), lambda i: (i, 0)),
                   pl.BlockSpec((tm, 128), lambda i: (i, 0))),
        compiler_params=pltpu.CompilerParams(
            dimension_semantics=("parallel",),
            vmem_limit_bytes=_VMEM_LIMIT),
    )(m4, x)


def _e2v_stage_kernel(m_ref, x_ref, o_ref):
    m4 = m_ref[...]
    e = m4.shape[0] * 8
    nc = m4.shape[1]
    z = x_ref[...]
    for j in range(nc):
        mj = m4[:, j].reshape(e, 128).astype(jnp.bfloat16)
        y = jax.lax.dot_general(mj, z, (((0,), (0,)), ((), ())),
                                preferred_element_type=jnp.float32)
        y = jnp.maximum(y, 0.0)
        sq = jnp.sum(y * y, axis=1, keepdims=True)
        o_ref[j * 128:(j + 1) * 128, :] = y * jax.lax.rsqrt(
            jnp.maximum(sq, 1e-24))


def _e2v_stage(m4, x, *, tm):
    nr8, ncol, _, _ = m4.shape
    e = nr8 * 8
    r = ncol * 128
    d = x.shape[1]
    return pl.pallas_call(
        _e2v_stage_kernel,
        out_shape=jax.ShapeDtypeStruct((r, d), jnp.float32),
        grid=(r // tm,),
        in_specs=[pl.BlockSpec((nr8, tm // 128, 8, 128),
                               lambda i: (0, i, 0, 0)),
                  pl.BlockSpec((e, d), lambda i: (0, 0))],
        out_specs=pl.BlockSpec((tm, d), lambda i: (i, 0)),
        compiler_params=pltpu.CompilerParams(
            dimension_semantics=("parallel",),
            vmem_limit_bytes=_VMEM_LIMIT),
    )(m4, x)


def _gcn_stage_kernel(m_ref, x_ref, o_ref):
    y = jnp.dot(_assemble_bf16(m_ref[...]), x_ref[...],
                preferred_element_type=jnp.float32)
    y = jnp.maximum(y, 0.0)
    sq = jnp.sum(y * y, axis=1, keepdims=True)
    o_ref[...] = y * jax.lax.rsqrt(jnp.maximum(sq, 1e-24))


def _gcn_stage(m4, x, *, tm):
    nr8, nc, _, _ = m4.shape
    r = nr8 * 8
    d = x.shape[1]
    return pl.pallas_call(
        _gcn_stage_kernel,
        out_shape=jax.ShapeDtypeStruct((r, d), jnp.float32),
        grid=(r // tm,),
        in_specs=[pl.BlockSpec((tm // 8, nc, 8, 128), lambda i: (i, 0, 0, 0)),
                  pl.BlockSpec((nc * 128, d), lambda i: (0, 0))],
        out_specs=pl.BlockSpec((tm, d), lambda i: (i, 0)),
        compiler_params=pltpu.CompilerParams(
            dimension_semantics=("parallel",),
            vmem_limit_bytes=_VMEM_LIMIT),
    )(m4, x)


def _final_stage_kernel(m_ref, h1b_ref, x_ref, xv_ref, h1_ref, o_ref):
    y = jnp.dot(_assemble_bf16(m_ref[...]), h1b_ref[...],
                preferred_element_type=jnp.float32)
    y = jnp.maximum(y, 0.0)
    sq = jnp.sum(y * y, axis=1, keepdims=True)
    h2 = y * jax.lax.rsqrt(jnp.maximum(sq, 1e-24))

    x = x_ref[...]
    a = xv_ref[...]
    b = h1_ref[...]
    tot = (jnp.sum(x * x, axis=1, keepdims=True)
           + jnp.sum(a * a, axis=1, keepdims=True)
           + jnp.sum(b * b, axis=1, keepdims=True)
           + jnp.sum(h2 * h2, axis=1, keepdims=True))
    inv = jax.lax.rsqrt(jnp.maximum(tot, 1e-24))
    d = x.shape[1]
    o_ref[:, 0:d] = x * inv
    o_ref[:, d:2 * d] = a * inv
    o_ref[:, 2 * d:3 * d] = b * inv
    o_ref[:, 3 * d:4 * d] = h2 * inv


def _final_stage(m4, h1b, x, xv, h1, *, tm):
    nr8, nc, _, _ = m4.shape
    r = nr8 * 8
    d = x.shape[1]
    row_spec = pl.BlockSpec((tm, d), lambda i: (i, 0))
    return pl.pallas_call(
        _final_stage_kernel,
        out_shape=jax.ShapeDtypeStruct((r, 4 * d), jnp.float32),
        grid=(r // tm,),
        in_specs=[pl.BlockSpec((tm // 8, nc, 8, 128), lambda i: (i, 0, 0, 0)),
                  pl.BlockSpec((nc * 128, d), lambda i: (0, 0)),
                  row_spec, row_spec, row_spec],
        out_specs=pl.BlockSpec((tm, 4 * d), lambda i: (i, 0)),
        compiler_params=pltpu.CompilerParams(
            dimension_semantics=("parallel",),
            vmem_limit_bytes=_VMEM_LIMIT),
    )(m4, h1b, x, xv, h1)


def _graph_forward(x, edge, size_v, size_e):
    row0, row1, row2 = edge[0], edge[1], edge[2]

    a_t4 = _scatter_counts_tiled(row1, row0, size_e, size_v)
    b4 = _scatter_counts_tiled(row2, row0, size_v, size_v)

    x32 = x.astype(jnp.float32)
    xb = x32.astype(jnp.bfloat16)

    z, zs = _v2e_stage(a_t4, xb, tm=_pick_tm(size_e, 512))
    deg_e = zs[:, 0]
    zb = (z * (1.0 / jnp.maximum(deg_e, 1.0))[:, None]).astype(jnp.bfloat16)

    x_v = _e2v_stage(a_t4, zb, tm=_pick_tm(size_v, 1024))
    h1 = _gcn_stage(b4, xb, tm=_pick_tm(size_v, 512))
    return _final_stage(b4, h1.astype(jnp.bfloat16), x32, x_v, h1,
                        tm=_pick_tm(size_v, 512))


@jax.jit
def kernel(x1, x2, edge1, edge2):
    y1 = _graph_forward(x1, edge1, 8192, 2048)
    y2 = _graph_forward(x2, edge2, 6144, 1536)
    return y1, y2
